# baseline (speedup 1.0000x reference)
import functools

import jax
import jax.numpy as jnp
from jax import lax
from jax.experimental import pallas as pl
from jax.experimental.pallas import tpu as pltpu
from jax.experimental.pallas import tpu_sc as plsc


def _sc_gather_body(tidx_hbm, table_hbm, out_hbm, tidx_v, tiles_v, sem):
    wid = lax.axis_index("s") * 2 + lax.axis_index("c")

    @pl.when(wid == 0)
    def _():
        pltpu.sync_copy(tidx_hbm, tidx_v)
        n = tiles_v.shape[0]
        for blk in range(n // 16):
            v = tidx_v[pl.ds(blk * 16, 16)]
            for i in range(16):
                pltpu.make_async_copy(
                    table_hbm.at[pl.ds(v[i], 1)],
                    tiles_v.at[pl.ds(blk * 16 + i, 1)],
                    sem,
                ).start()
        for i in range(n):
            pltpu.make_async_copy(
                table_hbm.at[pl.ds(0, 1)],
                tiles_v.at[pl.ds(0, 1)],
                sem,
            ).wait()
        pltpu.sync_copy(tiles_v, out_hbm)


def _matmul_body(sub_ref, xt_ref, w2_ref, b2_ref, out_ref, x2_ref):
    j = pl.program_id(0)
    n = xt_ref.shape[0]
    d = xt_ref.shape[2]

    @pl.when(j == 0)
    def _build_x2():
        x2_ref[...] = jnp.zeros_like(x2_ref)
        for i in range(n):
            row = xt_ref[i, pl.ds(sub_ref[i], 1), :].astype(jnp.bfloat16)
            x2_ref[pl.ds(i, 1), pl.ds(0, d)] = row
            x2_ref[pl.ds(n + i, 1), pl.ds(d, d)] = row

    res = jax.lax.dot_general(
        x2_ref[...], w2_ref[...],
        dimension_numbers=(((1,), (1,)), ((), ())),
        preferred_element_type=jnp.float32,
    )
    out_ref[:, 0, 0, :] = res[:n] + b2_ref[0, :][None, :]
    out_ref[:, 0, 1, :] = res[n:] + b2_ref[1, :][None, :]


def kernel(trg, enc_src, trg_mask, src_mask, emb_table, W, b):
    Bb, L = trg.shape
    V, D = emb_table.shape
    idx = trg.reshape(-1).astype(jnp.int32)
    n = idx.shape[0]
    tidx = idx // 8
    sub = idx % 8

    gather = functools.partial(
        pl.kernel,
        out_type=jax.ShapeDtypeStruct((n, 8, D), jnp.float32),
        mesh=plsc.VectorSubcoreMesh(core_axis_name="c", subcore_axis_name="s"),
        scratch_types=[
            pltpu.VMEM((n,), jnp.int32),
            pltpu.VMEM((n, 8, D), jnp.float32),
            pltpu.SemaphoreType.DMA,
        ],
        compiler_params=pltpu.CompilerParams(use_tc_tiling_on_sc=True),
    )(_sc_gather_body)
    xt = gather(tidx, emb_table.reshape(V // 8, 8, D))

    V2 = V // 2
    Wbf = W.astype(jnp.bfloat16)
    W2 = jnp.concatenate([Wbf[:V2], Wbf[V2:]], axis=1)
    b2 = b.reshape(2, V2)

    BV2 = 8192
    nv = pl.cdiv(V2, BV2)
    out = pl.pallas_call(
        _matmul_body,
        grid_spec=pltpu.PrefetchScalarGridSpec(
            num_scalar_prefetch=1,
            grid=(nv,),
            in_specs=[
                pl.BlockSpec((n, 8, D), lambda j, sub_ref: (0, 0, 0)),
                pl.BlockSpec((BV2, 2 * D), lambda j, sub_ref: (j, 0)),
                pl.BlockSpec((2, BV2), lambda j, sub_ref: (0, j)),
            ],
            out_specs=pl.BlockSpec((n, 1, 2, BV2),
                                   lambda j, sub_ref: (0, 0, 0, j)),
            scratch_shapes=[
                pltpu.VMEM((2 * n, 2 * D), jnp.bfloat16),
            ],
        ),
        out_shape=jax.ShapeDtypeStruct((n, 1, 2, V2), jnp.float32),
        compiler_params=pltpu.CompilerParams(
            dimension_semantics=("arbitrary",),
        ),
    )(sub, xt, W2, b2)
    return out.reshape(Bb, L, V)
